# baseline (speedup 1.0000x reference)
import functools

import jax
import jax.numpy as jnp
import numpy as np
from jax import lax
from jax.experimental import pallas as pl
from jax.experimental.pallas import tpu as pltpu
from jax.experimental.pallas import tpu_sc as plsc

_NW = 32


def _fps_body(n, ns, px_ref, py_ref, pz_ref, out_ref, pout_ref):
    rows = px_ref.shape[0]
    gidx = (lax.broadcasted_iota(jnp.int32, (rows, 128), 0) * 128
            + lax.broadcasted_iota(jnp.int32, (rows, 128), 1))
    lane = lax.broadcasted_iota(jnp.int32, (1, 128), 1)
    lane8 = lax.broadcasted_iota(jnp.int32, (1, 8), 1)
    px = px_ref[...]
    py = py_ref[...]
    pz = pz_ref[...]
    out_ref[0:1, :] = jnp.zeros((1, 1), jnp.int32)

    def _prow(sx, sy, sz):
        return jnp.where(lane8 == 0, sx,
                         jnp.where(lane8 == 1, sy,
                                   jnp.where(lane8 == 2, sz, 0.0)))

    def _sel(c, a, b):
        return tuple(jnp.where(c, u, w) for u, w in zip(a, b))

    def _argmax5(mind):
        t = (mind, gidx, px, py, pz)
        s = rows
        while s > 1:
            h = s // 2
            a = tuple(u[:h] for u in t)
            b = tuple(u[h:s] for u in t)
            t = _sel(a[0] >= b[0], a, b)
            s = h
        w = 64
        while w >= 1:
            a = tuple(u[:, :w] for u in t)
            b = tuple(u[:, w:2 * w] for u in t)
            t = _sel(a[0] >= b[0], a, b)
            w //= 2
        return t[1], t[2], t[3], t[4]

    def body(i, carry):
        mind0, sx, sy, sz = carry
        dx = px - sx
        dy = py - sy
        dz = pz - sz
        d = dx * dx + dy * dy + dz * dz
        mind = jnp.minimum(mind0, d)
        nxt, cx, cy, cz = _argmax5(mind)
        out_ref[pl.ds(i, 1), :] = nxt
        pout_ref[pl.ds(i, 1), :] = _prow(cx, cy, cz)
        return (mind, cx, cy, cz)

    pout_ref[0:1, :] = _prow(px[0:1, 0:1], py[0:1, 0:1], pz[0:1, 0:1])
    lax.fori_loop(1, ns, body,
                  (jnp.full((rows, 128), jnp.inf, jnp.float32),
                   px[0:1, 0:1], py[0:1, 0:1], pz[0:1, 0:1]))


def _fps(pos, ns):
    n = pos.shape[0]
    rows = n // 128
    px = pos[:, 0].reshape(rows, 128)
    py = pos[:, 1].reshape(rows, 128)
    pz = pos[:, 2].reshape(rows, 128)
    out, pout = pl.pallas_call(
        functools.partial(_fps_body, n, ns),
        out_shape=(jax.ShapeDtypeStruct((ns, 1), jnp.int32),
                   jax.ShapeDtypeStruct((ns, 8), jnp.float32)),
    )(px, py, pz)
    return out.reshape(ns), pout[:, :3]


def _knn_body(k, nsrc, pd_ref, psT_ref, idx_ref):
    pd = pd_ref[...]
    psT = psT_ref[...]
    dot = lax.dot_general(pd, psT, (((1,), (0,)), ((), ())),
                          preferred_element_type=jnp.float32)
    sd = jnp.sum(pd * pd, axis=1, keepdims=True)
    ss = jnp.sum(psT * psT, axis=0, keepdims=True)
    v = -((sd + ss) - 2.0 * dot)
    bd = v.shape[0]
    cols = lax.broadcasted_iota(jnp.int32, (bd, nsrc), 1)
    for t in range(k):
        m = jnp.max(v, axis=1, keepdims=True)
        it = jnp.min(jnp.where(v == m, cols, jnp.int32(nsrc)),
                     axis=1, keepdims=True)
        idx_ref[:, t:t + 1] = it
        v = jnp.where(cols == it, -jnp.inf, v)


def _knn(pos_src, pos_dst, k, bd=256):
    nd = pos_dst.shape[0]
    ns = pos_src.shape[0]
    pd = jnp.pad(pos_dst, ((0, 0), (0, 5)))
    psT = jnp.pad(pos_src, ((0, 0), (0, 5))).T
    idx = pl.pallas_call(
        functools.partial(_knn_body, k, ns),
        grid=(nd // bd,),
        in_specs=[pl.BlockSpec((bd, 8), lambda i: (i, 0)),
                  pl.BlockSpec((8, ns), lambda i: (0, 0))],
        out_specs=pl.BlockSpec((bd, k), lambda i: (i, 0)),
        out_shape=jax.ShapeDtypeStruct((nd, k), jnp.int32),
    )(pd, psT)
    return idx


def _pad_cols(a, m=128):
    d = a.shape[1]
    dp = ((d + m - 1) // m) * m
    return jnp.pad(a, ((0, 0), (0, dp - d)))


def _sc_gather(table, idx):
    V, D = table.shape
    B = idx.shape[0]
    bpw = B // _NW
    cs = min(bpw, 128 if D <= 512 else 64)
    nchunks = bpw // cs
    mesh = plsc.VectorSubcoreMesh(core_axis_name="c", subcore_axis_name="s")

    @functools.partial(
        pl.kernel,
        out_type=jax.ShapeDtypeStruct((B, D), jnp.float32),
        mesh=mesh,
        scratch_types=[
            pltpu.VMEM((bpw,), jnp.int32),
            pltpu.VMEM((cs, D), jnp.float32),
            pltpu.VMEM((cs, D), jnp.float32),
            pltpu.SemaphoreType.DMA,
            pltpu.SemaphoreType.DMA,
            pltpu.SemaphoreType.DMA,
            pltpu.SemaphoreType.DMA,
        ],
    )
    def gk(table_hbm, idx_hbm, out_hbm, idx_v, rows0, rows1, g0, g1, o0, o1):
        wid = lax.axis_index("s") * 2 + lax.axis_index("c")
        base = wid * bpw
        rows = (rows0, rows1)
        gsem = (g0, g1)
        osem = (o0, o1)
        pltpu.sync_copy(idx_hbm.at[pl.ds(base, bpw)], idx_v)
        gathers = [None, None]
        outs = [None, None]
        gathers[0] = pltpu.async_copy(
            table_hbm.at[idx_v.at[pl.ds(0, cs)]], rows[0], gsem[0])
        for c in range(nchunks):
            b = c % 2
            nb = (c + 1) % 2
            if c + 1 < nchunks:
                if outs[nb] is not None:
                    outs[nb].wait()
                gathers[nb] = pltpu.async_copy(
                    table_hbm.at[idx_v.at[pl.ds((c + 1) * cs, cs)]],
                    rows[nb], gsem[nb])
            gathers[b].wait()
            outs[b] = pltpu.async_copy(
                rows[b], out_hbm.at[pl.ds(base + c * cs, cs)], osem[b])
        for o in outs:
            if o is not None:
                o.wait()

    return gk(table, idx)


def _conv_body(k, nd, g_ref, pd_ref, w1_ref, b1_ref, w2_ref, b2_ref,
               wg_ref, bg_ref, out_ref):
    pd = pd_ref[...]
    acc = None
    for j in range(k):
        h = g_ref[j * nd:(j + 1) * nd, :] - pd
        h1 = lax.dot_general(h, w1_ref[...], (((1,), (0,)), ((), ())),
                             preferred_element_type=jnp.float32) + b1_ref[...]
        h1 = jnp.maximum(h1, 0.0)
        h2 = lax.dot_general(h1, w2_ref[...], (((1,), (0,)), ((), ())),
                             preferred_element_type=jnp.float32) + b2_ref[...]
        acc = h2 if acc is None else jnp.maximum(acc, h2)
    out_ref[...] = lax.dot_general(acc, wg_ref[...], (((1,), (0,)), ((), ())),
                                   preferred_element_type=jnp.float32) + bg_ref[...]


def _conv(g, pd_pad, p1, p2, pg, k, nd):
    dp = g.shape[1]
    w1 = jnp.pad(p1[0], ((0, dp - p1[0].shape[0]), (0, 0)))
    c1 = p1[0].shape[1]
    c2 = p2[0].shape[1]
    cg = pg[0].shape[1]
    out = pl.pallas_call(
        functools.partial(_conv_body, k, nd),
        out_shape=jax.ShapeDtypeStruct((nd, cg), jnp.float32),
    )(g, pd_pad, w1, p1[1].reshape(1, c1), p2[0], p2[1].reshape(1, c2),
      pg[0], pg[1].reshape(1, cg))
    return out


def _interp_body(k, gx_ref, gp_ref, pd_ref, out_ref):
    pd = pd_ref[...]
    num = None
    den = None
    for j in range(k):
        gpj = gp_ref[j]
        diff = pd - gpj
        d2 = jnp.sum(diff * diff, axis=1, keepdims=True)
        w = 1.0 / (d2 + 1e-16)
        contrib = w * gx_ref[j]
        num = contrib if num is None else num + contrib
        den = w if den is None else den + w
    out_ref[...] = num / den


def _interp(gx, gp, pos_dst, k, nd, bs):
    d = gx.shape[1]
    gx3 = gx.reshape(k, nd, d)
    gp3 = gp.reshape(k, nd, 8)
    pd = jnp.pad(pos_dst, ((0, 0), (0, 5)))
    return pl.pallas_call(
        functools.partial(_interp_body, k),
        grid=(nd // bs,),
        in_specs=[pl.BlockSpec((k, bs, d), lambda i: (0, i, 0)),
                  pl.BlockSpec((k, bs, 8), lambda i: (0, i, 0)),
                  pl.BlockSpec((bs, 8), lambda i: (i, 0))],
        out_specs=pl.BlockSpec((bs, d), lambda i: (i, 0)),
        out_shape=jax.ShapeDtypeStruct((nd, d), jnp.float32),
    )(gx3, gp3, pd)


def _mlp_body(h_ref, w1_ref, b1_ref, w2_ref, b2_ref, out_ref):
    h1 = lax.dot_general(h_ref[...], w1_ref[...], (((1,), (0,)), ((), ())),
                         preferred_element_type=jnp.float32) + b1_ref[...]
    h1 = jnp.maximum(h1, 0.0)
    out_ref[...] = lax.dot_general(h1, w2_ref[...], (((1,), (0,)), ((), ())),
                                   preferred_element_type=jnp.float32) + b2_ref[...]


def _mlp2(p1, p2, h):
    n = h.shape[0]
    c1 = p1[0].shape[1]
    c2 = p2[0].shape[1]
    return pl.pallas_call(
        _mlp_body,
        out_shape=jax.ShapeDtypeStruct((n, c2), jnp.float32),
    )(h, p1[0], p1[1].reshape(1, c1), p2[0], p2[1].reshape(1, c2))


def _fp1_heads_body(h_ref, w1_ref, b1_ref, w2_ref, b2_ref,
                    ws1_ref, bs1_ref, ws2_ref, bs2_ref,
                    wi1_ref, bi1_ref, wi2_ref, bi2_ref, sem_ref, inst_ref):
    mm = lambda a, b: lax.dot_general(a, b, (((1,), (0,)), ((), ())),
                                      preferred_element_type=jnp.float32)
    h1 = jnp.maximum(mm(h_ref[...], w1_ref[...]) + b1_ref[...], 0.0)
    xfp1 = mm(h1, w2_ref[...]) + b2_ref[...]
    hs = jnp.maximum(mm(xfp1, ws1_ref[...]) + bs1_ref[...], 0.0)
    sem_ref[...] = mm(hs, ws2_ref[...]) + bs2_ref[...]
    hi = jnp.maximum(mm(xfp1, wi1_ref[...]) + bi1_ref[...], 0.0)
    inst_ref[...] = mm(hi, wi2_ref[...]) + bi2_ref[...]


def kernel(x, pos, batch, params):
    n = pos.shape[0]
    feat = jnp.concatenate([x, pos], axis=1)

    _, pos1 = _fps(pos, n // 2)
    nd1 = n // 2
    knn1 = _knn(pos, pos1, 16)
    tab1 = _pad_cols(jnp.concatenate([feat, pos], axis=1))
    g1 = _sc_gather(tab1, knn1.T.reshape(-1))
    pd1 = jnp.pad(pos1, ((0, 0), (7, 118)))
    x1 = _conv(g1, pd1, params['sa1_l1'], params['sa1_l2'], params['sa1_g'],
               16, nd1)

    _, pos2 = _fps(pos1, nd1 // 4)
    nd2 = nd1 // 4
    knn2 = _knn(pos1, pos2, 16)
    tab2 = _pad_cols(jnp.concatenate([x1, pos1], axis=1))
    g2 = _sc_gather(tab2, knn2.T.reshape(-1))
    pd2 = jnp.pad(pos2, ((0, 0), (128, 125)))
    x2 = _conv(g2, pd2, params['sa2_l1'], params['sa2_l2'], params['sa2_g'],
               16, nd2)

    ki2 = _knn(pos2, pos1, 3)
    tabi2 = _pad_cols(jnp.concatenate([x2, pos2], axis=1))
    gi2 = _sc_gather(tabi2, ki2.T.reshape(-1))
    gx2 = gi2[:, :512]
    gp2 = jnp.pad(gi2[:, 512:515], ((0, 0), (0, 5)))
    xi2 = _interp(gx2, gp2, pos1, 3, nd1, 1024)
    xfp2 = _mlp2(params['fp2_1'], params['fp2_2'],
                 jnp.concatenate([xi2, x1], axis=1))

    ki1 = _knn(pos1, pos, 3)
    tabi1 = _pad_cols(jnp.concatenate([xfp2, pos1], axis=1))
    gi1 = _sc_gather(tabi1, ki1.T.reshape(-1))
    gx1 = gi1[:, :256]
    gp1 = jnp.pad(gi1[:, 256:259], ((0, 0), (0, 5)))
    xi1 = _interp(gx1, gp1, pos, 3, n, 2048)

    hin = jnp.concatenate([xi1, feat], axis=1)
    p = params
    sem, inst = pl.pallas_call(
        _fp1_heads_body,
        out_shape=(jax.ShapeDtypeStruct((n, 8), jnp.float32),
                   jax.ShapeDtypeStruct((n, 64), jnp.float32)),
    )(hin, p['fp1_1'][0], p['fp1_1'][1].reshape(1, -1),
      p['fp1_2'][0], p['fp1_2'][1].reshape(1, -1),
      p['sem1'][0], p['sem1'][1].reshape(1, -1),
      p['sem2'][0], p['sem2'][1].reshape(1, -1),
      p['inst1'][0], p['inst1'][1].reshape(1, -1),
      p['inst2'][0], p['inst2'][1].reshape(1, -1))
    return (sem, inst)


# speedup vs baseline: 1.2290x; 1.2290x over previous
import functools

import jax
import jax.numpy as jnp
import numpy as np
from jax import lax
from jax.experimental import pallas as pl
from jax.experimental.pallas import tpu as pltpu
from jax.experimental.pallas import tpu_sc as plsc

_NW = 32


def _fps_body(n, ns, px_ref, py_ref, pz_ref, out_ref, pout_ref):
    rows = px_ref.shape[0]
    gidx = (lax.broadcasted_iota(jnp.int32, (rows, 128), 0) * 128
            + lax.broadcasted_iota(jnp.int32, (rows, 128), 1))
    lane = lax.broadcasted_iota(jnp.int32, (1, 128), 1)
    lane8 = lax.broadcasted_iota(jnp.int32, (1, 8), 1)
    px = px_ref[...]
    py = py_ref[...]
    pz = pz_ref[...]
    out_ref[0:1, :] = jnp.zeros((1, 1), jnp.int32)

    def _prow(sx, sy, sz):
        return jnp.where(lane8 == 0, sx,
                         jnp.where(lane8 == 1, sy,
                                   jnp.where(lane8 == 2, sz, 0.0)))

    def _coords(last):
        row = last // 128
        col = last % 128
        lm = lane == col
        sx = jnp.sum(jnp.where(lm, px_ref[pl.ds(row, 1), :], 0.0))
        sy = jnp.sum(jnp.where(lm, py_ref[pl.ds(row, 1), :], 0.0))
        sz = jnp.sum(jnp.where(lm, pz_ref[pl.ds(row, 1), :], 0.0))
        return sx, sy, sz

    def body(i, carry):
        mind0, last = carry
        sx, sy, sz = _coords(last)
        pout_ref[pl.ds(i - 1, 1), :] = _prow(sx, sy, sz)
        dx = px - sx
        dy = py - sy
        dz = pz - sz
        d = dx * dx + dy * dy + dz * dz
        mind = jnp.minimum(mind0, d)
        m = jnp.max(mind)
        nxt = jnp.min(jnp.where(mind == m, gidx, jnp.int32(n)))
        out_ref[pl.ds(i, 1), :] = jnp.reshape(nxt, (1, 1))
        return (mind, nxt)

    _, fin = lax.fori_loop(1, ns, body,
                           (jnp.full((rows, 128), jnp.inf, jnp.float32),
                            jnp.int32(0)))
    fx, fy, fz = _coords(fin)
    pout_ref[pl.ds(ns - 1, 1), :] = _prow(fx, fy, fz)


def _fps(pos, ns):
    n = pos.shape[0]
    rows = n // 128
    px = pos[:, 0].reshape(rows, 128)
    py = pos[:, 1].reshape(rows, 128)
    pz = pos[:, 2].reshape(rows, 128)
    out, pout = pl.pallas_call(
        functools.partial(_fps_body, n, ns),
        out_shape=(jax.ShapeDtypeStruct((ns, 1), jnp.int32),
                   jax.ShapeDtypeStruct((ns, 8), jnp.float32)),
    )(px, py, pz)
    return out.reshape(ns), pout[:, :3]


def _knn_body(k, nsrc, pd_ref, psT_ref, idx_ref):
    pd = pd_ref[...]
    psT = psT_ref[...]
    dot = lax.dot_general(pd, psT, (((1,), (0,)), ((), ())),
                          preferred_element_type=jnp.float32)
    sd = jnp.sum(pd * pd, axis=1, keepdims=True)
    ss = jnp.sum(psT * psT, axis=0, keepdims=True)
    v = -((sd + ss) - 2.0 * dot)
    bd = v.shape[0]
    cols = lax.broadcasted_iota(jnp.int32, (bd, nsrc), 1)
    for t in range(k):
        m = jnp.max(v, axis=1, keepdims=True)
        it = jnp.min(jnp.where(v == m, cols, jnp.int32(nsrc)),
                     axis=1, keepdims=True)
        idx_ref[:, t:t + 1] = it
        v = jnp.where(cols == it, -jnp.inf, v)


def _knn(pos_src, pos_dst, k, bd=256):
    nd = pos_dst.shape[0]
    ns = pos_src.shape[0]
    pd = jnp.pad(pos_dst, ((0, 0), (0, 5)))
    psT = jnp.pad(pos_src, ((0, 0), (0, 5))).T
    idx = pl.pallas_call(
        functools.partial(_knn_body, k, ns),
        grid=(nd // bd,),
        in_specs=[pl.BlockSpec((bd, 8), lambda i: (i, 0)),
                  pl.BlockSpec((8, ns), lambda i: (0, 0))],
        out_specs=pl.BlockSpec((bd, k), lambda i: (i, 0)),
        out_shape=jax.ShapeDtypeStruct((nd, k), jnp.int32),
    )(pd, psT)
    return idx


def _pad_cols(a, m=128):
    d = a.shape[1]
    dp = ((d + m - 1) // m) * m
    return jnp.pad(a, ((0, 0), (0, dp - d)))


def _sc_gather(table, idx):
    V, D = table.shape
    B = idx.shape[0]
    bpw = B // _NW
    cs = min(bpw, 128 if D <= 512 else 64)
    nchunks = bpw // cs
    mesh = plsc.VectorSubcoreMesh(core_axis_name="c", subcore_axis_name="s")

    @functools.partial(
        pl.kernel,
        out_type=jax.ShapeDtypeStruct((B, D), jnp.float32),
        mesh=mesh,
        scratch_types=[
            pltpu.VMEM((bpw,), jnp.int32),
            pltpu.VMEM((cs, D), jnp.float32),
            pltpu.VMEM((cs, D), jnp.float32),
            pltpu.SemaphoreType.DMA,
            pltpu.SemaphoreType.DMA,
            pltpu.SemaphoreType.DMA,
            pltpu.SemaphoreType.DMA,
        ],
    )
    def gk(table_hbm, idx_hbm, out_hbm, idx_v, rows0, rows1, g0, g1, o0, o1):
        wid = lax.axis_index("s") * 2 + lax.axis_index("c")
        base = wid * bpw
        rows = (rows0, rows1)
        gsem = (g0, g1)
        osem = (o0, o1)
        pltpu.sync_copy(idx_hbm.at[pl.ds(base, bpw)], idx_v)
        gathers = [None, None]
        outs = [None, None]
        gathers[0] = pltpu.async_copy(
            table_hbm.at[idx_v.at[pl.ds(0, cs)]], rows[0], gsem[0])
        for c in range(nchunks):
            b = c % 2
            nb = (c + 1) % 2
            if c + 1 < nchunks:
                if outs[nb] is not None:
                    outs[nb].wait()
                gathers[nb] = pltpu.async_copy(
                    table_hbm.at[idx_v.at[pl.ds((c + 1) * cs, cs)]],
                    rows[nb], gsem[nb])
            gathers[b].wait()
            outs[b] = pltpu.async_copy(
                rows[b], out_hbm.at[pl.ds(base + c * cs, cs)], osem[b])
        for o in outs:
            if o is not None:
                o.wait()

    return gk(table, idx)


def _conv_body(k, nd, g_ref, pd_ref, w1_ref, b1_ref, w2_ref, b2_ref,
               wg_ref, bg_ref, out_ref):
    pd = pd_ref[...]
    acc = None
    for j in range(k):
        h = g_ref[j * nd:(j + 1) * nd, :] - pd
        h1 = lax.dot_general(h, w1_ref[...], (((1,), (0,)), ((), ())),
                             preferred_element_type=jnp.float32) + b1_ref[...]
        h1 = jnp.maximum(h1, 0.0)
        h2 = lax.dot_general(h1, w2_ref[...], (((1,), (0,)), ((), ())),
                             preferred_element_type=jnp.float32) + b2_ref[...]
        acc = h2 if acc is None else jnp.maximum(acc, h2)
    out_ref[...] = lax.dot_general(acc, wg_ref[...], (((1,), (0,)), ((), ())),
                                   preferred_element_type=jnp.float32) + bg_ref[...]


def _conv(g, pd_pad, p1, p2, pg, k, nd):
    dp = g.shape[1]
    w1 = jnp.pad(p1[0], ((0, dp - p1[0].shape[0]), (0, 0)))
    c1 = p1[0].shape[1]
    c2 = p2[0].shape[1]
    cg = pg[0].shape[1]
    out = pl.pallas_call(
        functools.partial(_conv_body, k, nd),
        out_shape=jax.ShapeDtypeStruct((nd, cg), jnp.float32),
    )(g, pd_pad, w1, p1[1].reshape(1, c1), p2[0], p2[1].reshape(1, c2),
      pg[0], pg[1].reshape(1, cg))
    return out


def _interp_body(k, gx_ref, gp_ref, pd_ref, out_ref):
    pd = pd_ref[...]
    num = None
    den = None
    for j in range(k):
        gpj = gp_ref[j]
        diff = pd - gpj
        d2 = jnp.sum(diff * diff, axis=1, keepdims=True)
        w = 1.0 / (d2 + 1e-16)
        contrib = w * gx_ref[j]
        num = contrib if num is None else num + contrib
        den = w if den is None else den + w
    out_ref[...] = num / den


def _interp(gx, gp, pos_dst, k, nd, bs):
    d = gx.shape[1]
    gx3 = gx.reshape(k, nd, d)
    gp3 = gp.reshape(k, nd, 8)
    pd = jnp.pad(pos_dst, ((0, 0), (0, 5)))
    return pl.pallas_call(
        functools.partial(_interp_body, k),
        grid=(nd // bs,),
        in_specs=[pl.BlockSpec((k, bs, d), lambda i: (0, i, 0)),
                  pl.BlockSpec((k, bs, 8), lambda i: (0, i, 0)),
                  pl.BlockSpec((bs, 8), lambda i: (i, 0))],
        out_specs=pl.BlockSpec((bs, d), lambda i: (i, 0)),
        out_shape=jax.ShapeDtypeStruct((nd, d), jnp.float32),
    )(gx3, gp3, pd)


def _mlp_body(h_ref, w1_ref, b1_ref, w2_ref, b2_ref, out_ref):
    h1 = lax.dot_general(h_ref[...], w1_ref[...], (((1,), (0,)), ((), ())),
                         preferred_element_type=jnp.float32) + b1_ref[...]
    h1 = jnp.maximum(h1, 0.0)
    out_ref[...] = lax.dot_general(h1, w2_ref[...], (((1,), (0,)), ((), ())),
                                   preferred_element_type=jnp.float32) + b2_ref[...]


def _mlp2(p1, p2, h):
    n = h.shape[0]
    c1 = p1[0].shape[1]
    c2 = p2[0].shape[1]
    return pl.pallas_call(
        _mlp_body,
        out_shape=jax.ShapeDtypeStruct((n, c2), jnp.float32),
    )(h, p1[0], p1[1].reshape(1, c1), p2[0], p2[1].reshape(1, c2))


def _fp1_heads_body(h_ref, w1_ref, b1_ref, w2_ref, b2_ref,
                    ws1_ref, bs1_ref, ws2_ref, bs2_ref,
                    wi1_ref, bi1_ref, wi2_ref, bi2_ref, sem_ref, inst_ref):
    mm = lambda a, b: lax.dot_general(a, b, (((1,), (0,)), ((), ())),
                                      preferred_element_type=jnp.float32)
    h1 = jnp.maximum(mm(h_ref[...], w1_ref[...]) + b1_ref[...], 0.0)
    xfp1 = mm(h1, w2_ref[...]) + b2_ref[...]
    hs = jnp.maximum(mm(xfp1, ws1_ref[...]) + bs1_ref[...], 0.0)
    sem_ref[...] = mm(hs, ws2_ref[...]) + bs2_ref[...]
    hi = jnp.maximum(mm(xfp1, wi1_ref[...]) + bi1_ref[...], 0.0)
    inst_ref[...] = mm(hi, wi2_ref[...]) + bi2_ref[...]


def kernel(x, pos, batch, params):
    n = pos.shape[0]
    feat = jnp.concatenate([x, pos], axis=1)

    _, pos1 = _fps(pos, n // 2)
    nd1 = n // 2
    knn1 = _knn(pos, pos1, 16)
    tab1 = _pad_cols(jnp.concatenate([feat, pos], axis=1))
    g1 = _sc_gather(tab1, knn1.T.reshape(-1))
    pd1 = jnp.pad(pos1, ((0, 0), (7, 118)))
    x1 = _conv(g1, pd1, params['sa1_l1'], params['sa1_l2'], params['sa1_g'],
               16, nd1)

    _, pos2 = _fps(pos1, nd1 // 4)
    nd2 = nd1 // 4
    knn2 = _knn(pos1, pos2, 16)
    tab2 = _pad_cols(jnp.concatenate([x1, pos1], axis=1))
    g2 = _sc_gather(tab2, knn2.T.reshape(-1))
    pd2 = jnp.pad(pos2, ((0, 0), (128, 125)))
    x2 = _conv(g2, pd2, params['sa2_l1'], params['sa2_l2'], params['sa2_g'],
               16, nd2)

    ki2 = _knn(pos2, pos1, 3)
    tabi2 = _pad_cols(jnp.concatenate([x2, pos2], axis=1))
    gi2 = _sc_gather(tabi2, ki2.T.reshape(-1))
    gx2 = gi2[:, :512]
    gp2 = jnp.pad(gi2[:, 512:515], ((0, 0), (0, 5)))
    xi2 = _interp(gx2, gp2, pos1, 3, nd1, 1024)
    xfp2 = _mlp2(params['fp2_1'], params['fp2_2'],
                 jnp.concatenate([xi2, x1], axis=1))

    ki1 = _knn(pos1, pos, 3)
    tabi1 = _pad_cols(jnp.concatenate([xfp2, pos1], axis=1))
    gi1 = _sc_gather(tabi1, ki1.T.reshape(-1))
    gx1 = gi1[:, :256]
    gp1 = jnp.pad(gi1[:, 256:259], ((0, 0), (0, 5)))
    xi1 = _interp(gx1, gp1, pos, 3, n, 2048)

    hin = jnp.concatenate([xi1, feat], axis=1)
    p = params
    sem, inst = pl.pallas_call(
        _fp1_heads_body,
        out_shape=(jax.ShapeDtypeStruct((n, 8), jnp.float32),
                   jax.ShapeDtypeStruct((n, 64), jnp.float32)),
    )(hin, p['fp1_1'][0], p['fp1_1'][1].reshape(1, -1),
      p['fp1_2'][0], p['fp1_2'][1].reshape(1, -1),
      p['sem1'][0], p['sem1'][1].reshape(1, -1),
      p['sem2'][0], p['sem2'][1].reshape(1, -1),
      p['inst1'][0], p['inst1'][1].reshape(1, -1),
      p['inst2'][0], p['inst2'][1].reshape(1, -1))
    return (sem, inst)


# speedup vs baseline: 1.3974x; 1.1370x over previous
import functools

import jax
import jax.numpy as jnp
import numpy as np
from jax import lax
from jax.experimental import pallas as pl
from jax.experimental.pallas import tpu as pltpu
from jax.experimental.pallas import tpu_sc as plsc

_NW = 32


def _fps_body(n, ns, px_ref, py_ref, pz_ref, out_ref, pout_ref):
    rows = px_ref.shape[0]
    gidx = (lax.broadcasted_iota(jnp.int32, (rows, 128), 0) * 128
            + lax.broadcasted_iota(jnp.int32, (rows, 128), 1))
    lane = lax.broadcasted_iota(jnp.int32, (1, 128), 1)
    lane8 = lax.broadcasted_iota(jnp.int32, (1, 8), 1)
    px = px_ref[...]
    py = py_ref[...]
    pz = pz_ref[...]
    out_ref[0:1, :] = jnp.zeros((1, 1), jnp.int32)

    def _prow(sx, sy, sz):
        return jnp.where(lane8 == 0, sx,
                         jnp.where(lane8 == 1, sy,
                                   jnp.where(lane8 == 2, sz, 0.0)))

    def _coords(last):
        row = last // 128
        col = last % 128
        lm = lane == col
        sx = jnp.sum(jnp.where(lm, px_ref[pl.ds(row, 1), :], 0.0))
        sy = jnp.sum(jnp.where(lm, py_ref[pl.ds(row, 1), :], 0.0))
        sz = jnp.sum(jnp.where(lm, pz_ref[pl.ds(row, 1), :], 0.0))
        return sx, sy, sz

    def body(i, carry):
        mind0, last = carry
        sx, sy, sz = _coords(last)
        pout_ref[pl.ds(i - 1, 1), :] = _prow(sx, sy, sz)
        dx = px - sx
        dy = py - sy
        dz = pz - sz
        d = dx * dx + dy * dy + dz * dz
        mind = jnp.minimum(mind0, d)
        nxt = jnp.argmax(mind).astype(jnp.int32)
        out_ref[pl.ds(i, 1), :] = jnp.reshape(nxt, (1, 1))
        return (mind, nxt)

    _, fin = lax.fori_loop(1, ns, body,
                           (jnp.full((rows, 128), jnp.inf, jnp.float32),
                            jnp.int32(0)))
    fx, fy, fz = _coords(fin)
    pout_ref[pl.ds(ns - 1, 1), :] = _prow(fx, fy, fz)


def _fps(pos, ns):
    n = pos.shape[0]
    rows = n // 128
    px = pos[:, 0].reshape(rows, 128)
    py = pos[:, 1].reshape(rows, 128)
    pz = pos[:, 2].reshape(rows, 128)
    out, pout = pl.pallas_call(
        functools.partial(_fps_body, n, ns),
        out_shape=(jax.ShapeDtypeStruct((ns, 1), jnp.int32),
                   jax.ShapeDtypeStruct((ns, 8), jnp.float32)),
    )(px, py, pz)
    return out.reshape(ns), pout[:, :3]


def _knn_body(k, nsrc, pd_ref, psT_ref, idx_ref):
    pd = pd_ref[...]
    psT = psT_ref[...]
    dot = lax.dot_general(pd, psT, (((1,), (0,)), ((), ())),
                          preferred_element_type=jnp.float32)
    sd = jnp.sum(pd * pd, axis=1, keepdims=True)
    ss = jnp.sum(psT * psT, axis=0, keepdims=True)
    v = -((sd + ss) - 2.0 * dot)
    bd = v.shape[0]
    cols = lax.broadcasted_iota(jnp.int32, (bd, nsrc), 1)
    for t in range(k):
        m = jnp.max(v, axis=1, keepdims=True)
        it = jnp.min(jnp.where(v == m, cols, jnp.int32(nsrc)),
                     axis=1, keepdims=True)
        idx_ref[:, t:t + 1] = it
        v = jnp.where(cols == it, -jnp.inf, v)


def _knn(pos_src, pos_dst, k, bd=256):
    nd = pos_dst.shape[0]
    ns = pos_src.shape[0]
    pd = jnp.pad(pos_dst, ((0, 0), (0, 5)))
    psT = jnp.pad(pos_src, ((0, 0), (0, 5))).T
    idx = pl.pallas_call(
        functools.partial(_knn_body, k, ns),
        grid=(nd // bd,),
        in_specs=[pl.BlockSpec((bd, 8), lambda i: (i, 0)),
                  pl.BlockSpec((8, ns), lambda i: (0, 0))],
        out_specs=pl.BlockSpec((bd, k), lambda i: (i, 0)),
        out_shape=jax.ShapeDtypeStruct((nd, k), jnp.int32),
    )(pd, psT)
    return idx


def _pad_cols(a, m=128):
    d = a.shape[1]
    dp = ((d + m - 1) // m) * m
    return jnp.pad(a, ((0, 0), (0, dp - d)))


def _sc_gather(table, idx):
    V, D = table.shape
    B = idx.shape[0]
    bpw = B // _NW
    cs = min(bpw, 128 if D <= 512 else 64)
    nchunks = bpw // cs
    mesh = plsc.VectorSubcoreMesh(core_axis_name="c", subcore_axis_name="s")

    @functools.partial(
        pl.kernel,
        out_type=jax.ShapeDtypeStruct((B, D), jnp.float32),
        mesh=mesh,
        scratch_types=[
            pltpu.VMEM((bpw,), jnp.int32),
            pltpu.VMEM((cs, D), jnp.float32),
            pltpu.VMEM((cs, D), jnp.float32),
            pltpu.SemaphoreType.DMA,
            pltpu.SemaphoreType.DMA,
            pltpu.SemaphoreType.DMA,
            pltpu.SemaphoreType.DMA,
        ],
    )
    def gk(table_hbm, idx_hbm, out_hbm, idx_v, rows0, rows1, g0, g1, o0, o1):
        wid = lax.axis_index("s") * 2 + lax.axis_index("c")
        base = wid * bpw
        rows = (rows0, rows1)
        gsem = (g0, g1)
        osem = (o0, o1)
        pltpu.sync_copy(idx_hbm.at[pl.ds(base, bpw)], idx_v)
        gathers = [None, None]
        outs = [None, None]
        gathers[0] = pltpu.async_copy(
            table_hbm.at[idx_v.at[pl.ds(0, cs)]], rows[0], gsem[0])
        for c in range(nchunks):
            b = c % 2
            nb = (c + 1) % 2
            if c + 1 < nchunks:
                if outs[nb] is not None:
                    outs[nb].wait()
                gathers[nb] = pltpu.async_copy(
                    table_hbm.at[idx_v.at[pl.ds((c + 1) * cs, cs)]],
                    rows[nb], gsem[nb])
            gathers[b].wait()
            outs[b] = pltpu.async_copy(
                rows[b], out_hbm.at[pl.ds(base + c * cs, cs)], osem[b])
        for o in outs:
            if o is not None:
                o.wait()

    return gk(table, idx)


def _conv_body(k, nd, g_ref, pd_ref, w1_ref, b1_ref, w2_ref, b2_ref,
               wg_ref, bg_ref, out_ref):
    pd = pd_ref[...]
    acc = None
    for j in range(k):
        h = g_ref[j * nd:(j + 1) * nd, :] - pd
        h1 = lax.dot_general(h, w1_ref[...], (((1,), (0,)), ((), ())),
                             preferred_element_type=jnp.float32) + b1_ref[...]
        h1 = jnp.maximum(h1, 0.0)
        h2 = lax.dot_general(h1, w2_ref[...], (((1,), (0,)), ((), ())),
                             preferred_element_type=jnp.float32) + b2_ref[...]
        acc = h2 if acc is None else jnp.maximum(acc, h2)
    out_ref[...] = lax.dot_general(acc, wg_ref[...], (((1,), (0,)), ((), ())),
                                   preferred_element_type=jnp.float32) + bg_ref[...]


def _conv(g, pd_pad, p1, p2, pg, k, nd):
    dp = g.shape[1]
    w1 = jnp.pad(p1[0], ((0, dp - p1[0].shape[0]), (0, 0)))
    c1 = p1[0].shape[1]
    c2 = p2[0].shape[1]
    cg = pg[0].shape[1]
    out = pl.pallas_call(
        functools.partial(_conv_body, k, nd),
        out_shape=jax.ShapeDtypeStruct((nd, cg), jnp.float32),
    )(g, pd_pad, w1, p1[1].reshape(1, c1), p2[0], p2[1].reshape(1, c2),
      pg[0], pg[1].reshape(1, cg))
    return out


def _interp_body(k, gx_ref, gp_ref, pd_ref, out_ref):
    pd = pd_ref[...]
    num = None
    den = None
    for j in range(k):
        gpj = gp_ref[j]
        diff = pd - gpj
        d2 = jnp.sum(diff * diff, axis=1, keepdims=True)
        w = 1.0 / (d2 + 1e-16)
        contrib = w * gx_ref[j]
        num = contrib if num is None else num + contrib
        den = w if den is None else den + w
    out_ref[...] = num / den


def _interp(gx, gp, pos_dst, k, nd, bs):
    d = gx.shape[1]
    gx3 = gx.reshape(k, nd, d)
    gp3 = gp.reshape(k, nd, 8)
    pd = jnp.pad(pos_dst, ((0, 0), (0, 5)))
    return pl.pallas_call(
        functools.partial(_interp_body, k),
        grid=(nd // bs,),
        in_specs=[pl.BlockSpec((k, bs, d), lambda i: (0, i, 0)),
                  pl.BlockSpec((k, bs, 8), lambda i: (0, i, 0)),
                  pl.BlockSpec((bs, 8), lambda i: (i, 0))],
        out_specs=pl.BlockSpec((bs, d), lambda i: (i, 0)),
        out_shape=jax.ShapeDtypeStruct((nd, d), jnp.float32),
    )(gx3, gp3, pd)


def _mlp_body(h_ref, w1_ref, b1_ref, w2_ref, b2_ref, out_ref):
    h1 = lax.dot_general(h_ref[...], w1_ref[...], (((1,), (0,)), ((), ())),
                         preferred_element_type=jnp.float32) + b1_ref[...]
    h1 = jnp.maximum(h1, 0.0)
    out_ref[...] = lax.dot_general(h1, w2_ref[...], (((1,), (0,)), ((), ())),
                                   preferred_element_type=jnp.float32) + b2_ref[...]


def _mlp2(p1, p2, h):
    n = h.shape[0]
    c1 = p1[0].shape[1]
    c2 = p2[0].shape[1]
    return pl.pallas_call(
        _mlp_body,
        out_shape=jax.ShapeDtypeStruct((n, c2), jnp.float32),
    )(h, p1[0], p1[1].reshape(1, c1), p2[0], p2[1].reshape(1, c2))


def _fp1_heads_body(h_ref, w1_ref, b1_ref, w2_ref, b2_ref,
                    ws1_ref, bs1_ref, ws2_ref, bs2_ref,
                    wi1_ref, bi1_ref, wi2_ref, bi2_ref, sem_ref, inst_ref):
    mm = lambda a, b: lax.dot_general(a, b, (((1,), (0,)), ((), ())),
                                      preferred_element_type=jnp.float32)
    h1 = jnp.maximum(mm(h_ref[...], w1_ref[...]) + b1_ref[...], 0.0)
    xfp1 = mm(h1, w2_ref[...]) + b2_ref[...]
    hs = jnp.maximum(mm(xfp1, ws1_ref[...]) + bs1_ref[...], 0.0)
    sem_ref[...] = mm(hs, ws2_ref[...]) + bs2_ref[...]
    hi = jnp.maximum(mm(xfp1, wi1_ref[...]) + bi1_ref[...], 0.0)
    inst_ref[...] = mm(hi, wi2_ref[...]) + bi2_ref[...]


def kernel(x, pos, batch, params):
    n = pos.shape[0]
    feat = jnp.concatenate([x, pos], axis=1)

    _, pos1 = _fps(pos, n // 2)
    nd1 = n // 2
    knn1 = _knn(pos, pos1, 16)
    tab1 = _pad_cols(jnp.concatenate([feat, pos], axis=1))
    g1 = _sc_gather(tab1, knn1.T.reshape(-1))
    pd1 = jnp.pad(pos1, ((0, 0), (7, 118)))
    x1 = _conv(g1, pd1, params['sa1_l1'], params['sa1_l2'], params['sa1_g'],
               16, nd1)

    _, pos2 = _fps(pos1, nd1 // 4)
    nd2 = nd1 // 4
    knn2 = _knn(pos1, pos2, 16)
    tab2 = _pad_cols(jnp.concatenate([x1, pos1], axis=1))
    g2 = _sc_gather(tab2, knn2.T.reshape(-1))
    pd2 = jnp.pad(pos2, ((0, 0), (128, 125)))
    x2 = _conv(g2, pd2, params['sa2_l1'], params['sa2_l2'], params['sa2_g'],
               16, nd2)

    ki2 = _knn(pos2, pos1, 3)
    tabi2 = _pad_cols(jnp.concatenate([x2, pos2], axis=1))
    gi2 = _sc_gather(tabi2, ki2.T.reshape(-1))
    gx2 = gi2[:, :512]
    gp2 = jnp.pad(gi2[:, 512:515], ((0, 0), (0, 5)))
    xi2 = _interp(gx2, gp2, pos1, 3, nd1, 1024)
    xfp2 = _mlp2(params['fp2_1'], params['fp2_2'],
                 jnp.concatenate([xi2, x1], axis=1))

    ki1 = _knn(pos1, pos, 3)
    tabi1 = _pad_cols(jnp.concatenate([xfp2, pos1], axis=1))
    gi1 = _sc_gather(tabi1, ki1.T.reshape(-1))
    gx1 = gi1[:, :256]
    gp1 = jnp.pad(gi1[:, 256:259], ((0, 0), (0, 5)))
    xi1 = _interp(gx1, gp1, pos, 3, n, 2048)

    hin = jnp.concatenate([xi1, feat], axis=1)
    p = params
    sem, inst = pl.pallas_call(
        _fp1_heads_body,
        out_shape=(jax.ShapeDtypeStruct((n, 8), jnp.float32),
                   jax.ShapeDtypeStruct((n, 64), jnp.float32)),
    )(hin, p['fp1_1'][0], p['fp1_1'][1].reshape(1, -1),
      p['fp1_2'][0], p['fp1_2'][1].reshape(1, -1),
      p['sem1'][0], p['sem1'][1].reshape(1, -1),
      p['sem2'][0], p['sem2'][1].reshape(1, -1),
      p['inst1'][0], p['inst1'][1].reshape(1, -1),
      p['inst2'][0], p['inst2'][1].reshape(1, -1))
    return (sem, inst)
